# baseline (speedup 1.0000x reference)
import functools

import jax
import jax.numpy as jnp
from jax import lax
from jax.experimental import pallas as pl
from jax.experimental.pallas import tpu as pltpu
from jax.experimental.pallas import tpu_sc as plsc

B, S, H, V = 4, 2048, 1024, 100000
NC, NS, L = 2, 16, 16
NW = NC * NS
SBLK = S // NW
C = 16
NCH = B * SBLK // C

_mesh = plsc.VectorSubcoreMesh(core_axis_name="c", subcore_axis_name="s")


@functools.partial(
    pl.kernel,
    mesh=_mesh,
    out_type=jax.ShapeDtypeStruct((B * S, H), jnp.float32),
    scratch_types=[
        pltpu.VMEM((B * SBLK,), jnp.int32),
        pltpu.VMEM((B * SBLK,), jnp.int32),
        pltpu.VMEM((C, H), jnp.float32),
        pltpu.VMEM((C, H), jnp.float32),
        pltpu.VMEM((C, H), jnp.float32),
        pltpu.VMEM((C, H), jnp.float32),
        pltpu.SemaphoreType.DMA,
        pltpu.SemaphoreType.DMA,
        pltpu.SemaphoreType.DMA,
        pltpu.SemaphoreType.DMA,
        pltpu.SemaphoreType.DMA,
        pltpu.SemaphoreType.DMA,
    ],
)
def _emb_kernel(ids_hbm, tt_hbm, w_hbm, p_hbm, t_hbm, out_hbm,
                idv, ttv, wbuf0, wbuf1, tbuf0, tbuf1,
                sem_w0, sem_w1, sem_t0, sem_t1, sem_o0, sem_o1):
    wid = lax.axis_index("s") * NC + lax.axis_index("c")
    s0 = wid * SBLK
    wbufs = (wbuf0, wbuf1)
    tbufs = (tbuf0, tbuf1)
    sems_w = (sem_w0, sem_w1)
    sems_t = (sem_t0, sem_t1)
    sems_o = (sem_o0, sem_o1)
    for b in range(B):
        pltpu.sync_copy(ids_hbm.at[pl.ds(b * S + s0, SBLK)],
                        idv.at[pl.ds(b * SBLK, SBLK)])
        pltpu.sync_copy(tt_hbm.at[pl.ds(b * S + s0, SBLK)],
                        ttv.at[pl.ds(b * SBLK, SBLK)])

    gathers = [None, None]
    stores = [None, None]

    def launch(i):
        slot = i % 2
        if stores[slot] is not None:
            stores[slot].wait()
            stores[slot] = None
        cp_w = pltpu.async_copy(w_hbm.at[idv.at[pl.ds(i * C, C)]],
                                wbufs[slot], sems_w[slot])
        cp_t = pltpu.async_copy(t_hbm.at[ttv.at[pl.ds(i * C, C)]],
                                tbufs[slot], sems_t[slot])
        gathers[slot] = (cp_w, cp_t)

    launch(0)
    for i in range(NCH):
        if i + 1 < NCH:
            launch(i + 1)
        slot = i % 2
        b, c = divmod(i, SBLK // C)
        cp_w, cp_t = gathers[slot]
        cp_w.wait()
        cp_t.wait()
        off = b * S + s0 + c * C
        stores[slot] = pltpu.async_copy(wbufs[slot],
                                        out_hbm.at[pl.ds(off, C)],
                                        sems_o[slot])
    for slot in range(2):
        if stores[slot] is not None:
            stores[slot].wait()


def kernel(input_ids, token_type_ids, word_embeddings, position_embeddings,
           token_type_embeddings):
    ids = input_ids.reshape(-1).astype(jnp.int32)
    tt = token_type_ids.reshape(-1).astype(jnp.int32)
    out = _emb_kernel(ids, tt, word_embeddings, position_embeddings,
                      token_type_embeddings)
    return out.reshape(B, S, H)


# speedup vs baseline: 1.6684x; 1.6684x over previous
import functools

import jax
import jax.numpy as jnp
from jax import lax
from jax.experimental import pallas as pl
from jax.experimental.pallas import tpu as pltpu
from jax.experimental.pallas import tpu_sc as plsc

B, S, H, V = 4, 2048, 1024, 100000
NC, NS, L = 2, 16, 16
NW = NC * NS
SBLK = S // NW
C = 16
NCC = SBLK // C
NCH = B * NCC
JW = H // L

_mesh = plsc.VectorSubcoreMesh(core_axis_name="c", subcore_axis_name="s")


@functools.partial(
    pl.kernel,
    mesh=_mesh,
    compiler_params=pltpu.CompilerParams(needs_layout_passes=False),
    out_type=jax.ShapeDtypeStruct((B * S, H), jnp.float32),
    scratch_types=[
        pltpu.VMEM((SBLK, H), jnp.float32),
        pltpu.VMEM((2, H), jnp.float32),
        pltpu.VMEM((B * SBLK,), jnp.int32),
        pltpu.VMEM((B * SBLK,), jnp.int32),
        pltpu.VMEM((C, H), jnp.float32),
        pltpu.VMEM((C, H), jnp.float32),
        pltpu.SemaphoreType.DMA,
        pltpu.SemaphoreType.DMA,
        pltpu.SemaphoreType.DMA,
        pltpu.SemaphoreType.DMA,
    ],
)
def _emb_kernel(ids_hbm, tt_hbm, w_hbm, p_hbm, t_hbm, out_hbm,
                pbuf, tvm, idv, ttv, wbuf0, wbuf1,
                sem_w0, sem_w1, sem_o0, sem_o1):
    wid = lax.axis_index("s") * NC + lax.axis_index("c")
    s0 = wid * SBLK
    wbufs = (wbuf0, wbuf1)
    sems_w = (sem_w0, sem_w1)
    sems_o = (sem_o0, sem_o1)

    pltpu.sync_copy(p_hbm.at[pl.ds(s0, SBLK)], pbuf)
    pltpu.sync_copy(t_hbm, tvm)
    for b in range(B):
        pltpu.sync_copy(ids_hbm.at[pl.ds(b * S + s0, SBLK)],
                        idv.at[pl.ds(b * SBLK, SBLK)])
        pltpu.sync_copy(tt_hbm.at[pl.ds(b * S + s0, SBLK)],
                        ttv.at[pl.ds(b * SBLK, SBLK)])

    gathers = [None, None]
    stores = [None, None]

    def launch(i):
        slot = i % 2
        if stores[slot] is not None:
            stores[slot].wait()
            stores[slot] = None
        gathers[slot] = pltpu.async_copy(w_hbm.at[idv.at[pl.ds(i * C, C)]],
                                         wbufs[slot], sems_w[slot])

    base_iota = lax.iota(jnp.int32, L)

    launch(0)
    for i in range(NCH):
        if i + 1 < NCH:
            launch(i + 1)
        slot = i % 2
        b, c = divmod(i, NCC)
        gathers[slot].wait()
        wb = wbufs[slot]

        def row_body(r, _, wb=wb, c=c, i=i):
            ttr = plsc.load_gather(ttv, [jnp.full((L,), i * C + r,
                                                  jnp.int32)])

            def col_body(j, _):
                for k in range(4):
                    off = j * (4 * L) + k * L
                    col = pl.ds(off, L)
                    tv = plsc.load_gather(tvm, [ttr, base_iota + off])
                    wb[r, col] = (wb[r, col] + pbuf[c * C + r, col] + tv)
                return 0

            lax.fori_loop(0, JW // 4, col_body, 0, unroll=False)
            return 0

        lax.fori_loop(0, C, row_body, 0, unroll=False)
        off = b * S + s0 + c * C
        stores[slot] = pltpu.async_copy(wb, out_hbm.at[pl.ds(off, C)],
                                        sems_o[slot])
    for slot in range(2):
        if stores[slot] is not None:
            stores[slot].wait()


def kernel(input_ids, token_type_ids, word_embeddings, position_embeddings,
           token_type_embeddings):
    ids = input_ids.reshape(-1).astype(jnp.int32)
    tt = token_type_ids.reshape(-1).astype(jnp.int32)
    out = _emb_kernel(ids, tt, word_embeddings, position_embeddings,
                      token_type_embeddings)
    return out.reshape(B, S, H)
